# baseline (speedup 1.0000x reference)
import functools

import jax
import jax.numpy as jnp
from jax import lax
from jax.experimental import pallas as pl
from jax.experimental.pallas import tpu as pltpu
from jax.experimental.pallas import tpu_sc as plsc

BATCH = 16384
DIM = 64
L = 16

_info = plsc.get_sparse_core_info()
NC, NS = _info.num_cores, _info.num_subcores
NW = NC * NS
B_PER_W = BATCH // NW
CHUNK = 256
N_CHUNKS = B_PER_W // CHUNK


def _score_groups(srows, rrows, orows, trows, outv, out_base):
    lanes = lax.iota(jnp.int32, L)

    def group_body(g, _):
        rowv = lanes + g * L

        def d_body(d, acc):
            col = jnp.full((L,), 0, jnp.int32) + d
            sv = plsc.load_gather(srows, [rowv, col])
            rv = plsc.load_gather(rrows, [rowv, col])
            ov = plsc.load_gather(orows, [rowv, col])
            tv = plsc.load_gather(trows, [rowv, col])
            diff = sv + rv + tv - ov
            return acc + diff * diff

        acc = lax.fori_loop(0, DIM, d_body, jnp.zeros((L,), jnp.float32))
        seed = jnp.int32(0x5F3759DF) - (plsc.bitcast(acc, jnp.int32) >> 1)
        y = plsc.bitcast(seed, jnp.float32)
        half = acc * jnp.float32(0.5)
        for _i in range(3):
            y = y * (jnp.float32(1.5) - half * y * y)
        outv[pl.ds(out_base + g * L, L)] = -(acc * y)
        return 0

    lax.fori_loop(0, CHUNK // L, group_body, 0)


def _body(s_id, r_id, o_id, t_id, ent, rel, tim, out,
          sidx, ridx, oidx, tidx, srows, rrows, orows, trows, outv, sem):
    wid = lax.axis_index("s") * NC + lax.axis_index("c")
    base = wid * B_PER_W
    pltpu.sync_copy(s_id.at[pl.ds(base, B_PER_W)], sidx)
    pltpu.sync_copy(r_id.at[pl.ds(base, B_PER_W)], ridx)
    pltpu.sync_copy(o_id.at[pl.ds(base, B_PER_W)], oidx)
    pltpu.sync_copy(t_id.at[pl.ds(base, B_PER_W)], tidx)
    for c in range(N_CHUNKS):
        cb = c * CHUNK
        cps = pltpu.async_copy(ent.at[sidx.at[pl.ds(cb, CHUNK)]], srows, sem)
        cpr = pltpu.async_copy(rel.at[ridx.at[pl.ds(cb, CHUNK)]], rrows, sem)
        cpo = pltpu.async_copy(ent.at[oidx.at[pl.ds(cb, CHUNK)]], orows, sem)
        cpt = pltpu.async_copy(tim.at[tidx.at[pl.ds(cb, CHUNK)]], trows, sem)
        cps.wait()
        cpr.wait()
        cpo.wait()
        cpt.wait()
        _score_groups(srows, rrows, orows, trows, outv, cb)
    pltpu.sync_copy(outv, out.at[pl.ds(base, B_PER_W)])


_sc_call = functools.partial(
    pl.kernel,
    mesh=plsc.VectorSubcoreMesh(core_axis_name="c", subcore_axis_name="s"),
    out_type=jax.ShapeDtypeStruct((BATCH,), jnp.float32),
    compiler_params=pltpu.CompilerParams(use_tc_tiling_on_sc=False,
                                         needs_layout_passes=False),
    scratch_types=[
        pltpu.VMEM((B_PER_W,), jnp.int32),
        pltpu.VMEM((B_PER_W,), jnp.int32),
        pltpu.VMEM((B_PER_W,), jnp.int32),
        pltpu.VMEM((B_PER_W,), jnp.int32),
        pltpu.VMEM((CHUNK, DIM), jnp.float32),
        pltpu.VMEM((CHUNK, DIM), jnp.float32),
        pltpu.VMEM((CHUNK, DIM), jnp.float32),
        pltpu.VMEM((CHUNK, DIM), jnp.float32),
        pltpu.VMEM((B_PER_W,), jnp.float32),
        pltpu.SemaphoreType.DMA,
    ],
)(_body)


def kernel(s_id, r_id, o_id, t_id, entities, relations, times):
    return _sc_call(s_id.astype(jnp.int32), r_id.astype(jnp.int32),
                    o_id.astype(jnp.int32), t_id.astype(jnp.int32),
                    entities, relations, times)


# speedup vs baseline: 1.3488x; 1.3488x over previous
import functools

import jax
import jax.numpy as jnp
from jax import lax
from jax.experimental import pallas as pl
from jax.experimental.pallas import tpu as pltpu
from jax.experimental.pallas import tpu_sc as plsc

BATCH = 16384
DIM = 64
L = 16
NTAB = 1000

_info = plsc.get_sparse_core_info()
NC, NS = _info.num_cores, _info.num_subcores
NW = NC * NS
B_PER_W = BATCH // NW
CHUNK = 64
N_CHUNKS = B_PER_W // CHUNK


def _body(s_id, r_id, o_id, t_id, ent, rel_flat, tim_flat, out,
          sidx, ridx, oidx, tidx, tab, rt, srow, orow, outv, sem, semt):
    wid = lax.axis_index("s") * NC + lax.axis_index("c")
    base = wid * B_PER_W
    lanes = lax.iota(jnp.int32, L)

    cp_tab = pltpu.async_copy(rel_flat, tab, semt)
    pltpu.sync_copy(s_id.at[pl.ds(base, B_PER_W)], sidx)
    pltpu.sync_copy(r_id.at[pl.ds(base, B_PER_W)], ridx)
    pltpu.sync_copy(o_id.at[pl.ds(base, B_PER_W)], oidx)
    pltpu.sync_copy(t_id.at[pl.ds(base, B_PER_W)], tidx)
    cp_tab.wait()

    def rel_group(g, _):
        rowv = lanes + g * L
        tv = ridx[pl.ds(g * L, L)] * DIM
        dstv = rowv * DIM

        def d_body(d, _):
            v = plsc.load_gather(tab, [tv + d])
            plsc.store_scatter(rt, [dstv + d], v)
            return 0

        lax.fori_loop(0, DIM, d_body, 0)
        return 0

    lax.fori_loop(0, B_PER_W // L, rel_group, 0)

    pltpu.sync_copy(tim_flat, tab)

    def tim_group(g, _):
        rowv = lanes + g * L
        tv = tidx[pl.ds(g * L, L)] * DIM
        dstv = rowv * DIM

        def d_body(d, _):
            v = plsc.load_gather(tab, [tv + d])
            plsc.addupdate_scatter(rt, [dstv + d], v)
            return 0

        lax.fori_loop(0, DIM, d_body, 0)
        return 0

    lax.fori_loop(0, B_PER_W // L, tim_group, 0)

    for c in range(N_CHUNKS):
        cb = c * CHUNK

        def fetch(g, _):
            sv_idx = sidx[pl.ds(cb + g * L, L)]
            ov_idx = oidx[pl.ds(cb + g * L, L)]
            j0 = g * L
            for k in range(L):
                si = sv_idx[k]
                oi = ov_idx[k]
                pltpu.async_copy(ent.at[pl.ds(si, 1)],
                                 srow.at[pl.ds(j0 + k, 1)], sem)
                pltpu.async_copy(ent.at[pl.ds(oi, 1)],
                                 orow.at[pl.ds(j0 + k, 1)], sem)
            return 0

        lax.fori_loop(0, CHUNK // L, fetch, 0)
        pltpu.make_async_copy(ent.at[pl.ds(0, CHUNK)], srow, sem).wait()
        pltpu.make_async_copy(ent.at[pl.ds(0, CHUNK)], orow, sem).wait()

        def score_group(g, _):
            lrow = lanes + g * L
            grow = (lrow + cb) * DIM

            def d_body(d, acc):
                col = jnp.full((L,), 0, jnp.int32) + d
                sv = plsc.load_gather(srow, [lrow, col])
                ov = plsc.load_gather(orow, [lrow, col])
                rtv = plsc.load_gather(rt, [grow + d])
                diff = sv + rtv - ov
                return acc + diff * diff

            acc = lax.fori_loop(0, DIM, d_body, jnp.zeros((L,), jnp.float32))
            seed = jnp.int32(0x5F3759DF) - (plsc.bitcast(acc, jnp.int32) >> 1)
            y = plsc.bitcast(seed, jnp.float32)
            half = acc * jnp.float32(0.5)
            for _i in range(3):
                y = y * (jnp.float32(1.5) - half * y * y)
            outv[pl.ds(cb + g * L, L)] = -(acc * y)
            return 0

        lax.fori_loop(0, CHUNK // L, score_group, 0)

    pltpu.sync_copy(outv, out.at[pl.ds(base, B_PER_W)])


_sc_call = functools.partial(
    pl.kernel,
    mesh=plsc.VectorSubcoreMesh(core_axis_name="c", subcore_axis_name="s"),
    out_type=jax.ShapeDtypeStruct((BATCH,), jnp.float32),
    compiler_params=pltpu.CompilerParams(needs_layout_passes=False),
    scratch_types=[
        pltpu.VMEM((B_PER_W,), jnp.int32),
        pltpu.VMEM((B_PER_W,), jnp.int32),
        pltpu.VMEM((B_PER_W,), jnp.int32),
        pltpu.VMEM((B_PER_W,), jnp.int32),
        pltpu.VMEM((NTAB * DIM,), jnp.float32),
        pltpu.VMEM((B_PER_W * DIM,), jnp.float32),
        pltpu.VMEM((CHUNK, DIM), jnp.float32),
        pltpu.VMEM((CHUNK, DIM), jnp.float32),
        pltpu.VMEM((B_PER_W,), jnp.float32),
        pltpu.SemaphoreType.DMA,
        pltpu.SemaphoreType.DMA,
    ],
)(_body)


def kernel(s_id, r_id, o_id, t_id, entities, relations, times):
    return _sc_call(s_id.astype(jnp.int32), r_id.astype(jnp.int32),
                    o_id.astype(jnp.int32), t_id.astype(jnp.int32),
                    entities, relations.reshape(-1), times.reshape(-1))
